# baseline (speedup 1.0000x reference)
import functools

import jax
import jax.numpy as jnp
from jax import lax
from jax.experimental import pallas as pl
from jax.experimental.pallas import tpu as pltpu
from jax.experimental.pallas import tpu_sc as plsc

BATCH = 4096
HIST = 200
D = 32
OUT_L = HIST + 1
NC = 2
NS = 16
NW = NC * NS
B_PER_W = BATCH // NW
LANES = 16


def _sc_embed(val, table, global_token):
    mesh = plsc.VectorSubcoreMesh(core_axis_name="c", subcore_axis_name="s")

    @functools.partial(
        pl.kernel,
        mesh=mesh,
        out_type=jax.ShapeDtypeStruct((OUT_L, D // 8, NW, 8, B_PER_W), jnp.float32),
        scratch_types=[
            pltpu.VMEM((B_PER_W, HIST), jnp.int32),
            pltpu.VMEM((HIST * B_PER_W,), jnp.int32),
            pltpu.VMEM((B_PER_W, D), jnp.float32),
            pltpu.VMEM((B_PER_W, D), jnp.float32),
            pltpu.VMEM((B_PER_W, D), jnp.float32),
            pltpu.VMEM((B_PER_W, D), jnp.float32),
            pltpu.VMEM((D, B_PER_W), jnp.float32),
            pltpu.VMEM((D, B_PER_W), jnp.float32),
            pltpu.VMEM((D,), jnp.float32),
            pltpu.SemaphoreType.DMA,
            pltpu.SemaphoreType.DMA,
            pltpu.SemaphoreType.DMA,
            pltpu.SemaphoreType.DMA,
            pltpu.SemaphoreType.DMA,
            pltpu.SemaphoreType.DMA,
        ],
        compiler_params=pltpu.CompilerParams(
            use_tc_tiling_on_sc=False, needs_layout_passes=False),
    )
    def k(val_hbm, table_hbm, gt_hbm, out_hbm,
          idx_raw, idx_t, r0, r1, r2, r3, t0, t1, gt_v,
          g0, g1, g2, g3, w0, w1):
        rows = (r0, r1, r2, r3)
        gsems = (g0, g1, g2, g3)
        tils = (t0, t1)
        wsems = (w0, w1)

        wid = lax.axis_index("s") * NC + lax.axis_index("c")
        b0 = wid * B_PER_W

        pltpu.sync_copy(val_hbm.at[pl.ds(b0, B_PER_W)], idx_raw)
        pltpu.sync_copy(gt_hbm.at[0], gt_v)

        iota = lax.iota(jnp.int32, LANES)
        zeros = (iota + wid) // (LANES + NW)

        @plsc.parallel_loop(0, HIST, unroll=2)
        def idx_body(l):
            for j in range(B_PER_W // LANES):
                g = plsc.load_gather(idx_raw, [iota + j * LANES, zeros + l])
                idx_t[pl.ds(l * B_PER_W + j * LANES, LANES)] = g

        for c in range(D):
            bc = plsc.load_gather(gt_v, [zeros + c])
            for j in range(B_PER_W // LANES):
                t0[c, pl.ds(j * LANES, LANES)] = bc
        for ci in range(D // 8):
            pltpu.sync_copy(t0.at[pl.ds(ci * 8, 8)], out_hbm.at[0, ci, wid])

        def g_copy(l, rbuf, gsem):
            return pltpu.make_async_copy(
                table_hbm.at[idx_t.at[pl.ds(l * B_PER_W, B_PER_W)]], rbuf, gsem)

        def w_copy(tbuf, wsem, ci, p):
            return pltpu.make_async_copy(
                tbuf.at[pl.ds(ci * 8, 8)], out_hbm.at[p, ci, wid], wsem)

        def transpose(rbuf, tbuf):
            @plsc.parallel_loop(0, B_PER_W // LANES, unroll=4)
            def j_body(j):
                for c in range(D):
                    g = plsc.load_gather(rbuf, [iota + j * LANES, zeros + c])
                    tbuf[c, pl.ds(j * LANES, LANES)] = g

        def slot(l, kslot, drain_w, prefetch):
            rbuf, gsem = rows[kslot % 4], gsems[kslot % 4]
            tbuf, wsem = tils[kslot % 2], wsems[kslot % 2]
            g_copy(l, rbuf, gsem).wait()
            if drain_w:
                for ci in range(D // 8):
                    w_copy(tbuf, wsem, ci, l - 1).wait()
            transpose(rbuf, tbuf)
            for ci in range(D // 8):
                w_copy(tbuf, wsem, ci, l + 1).start()
            if prefetch:
                g_copy(l + 4, rbuf, gsem).start()

        for l in range(4):
            g_copy(l, rows[l], gsems[l]).start()
        slot(0, 0, False, True)
        slot(1, 1, False, True)
        slot(2, 2, True, True)
        slot(3, 3, True, True)

        def body(q, carry):
            l = 4 * q
            slot(l, 0, True, True)
            slot(l + 1, 1, True, True)
            slot(l + 2, 2, True, True)
            slot(l + 3, 3, True, True)
            return carry

        lax.fori_loop(1, HIST // 4 - 1, body, 0)
        lf = HIST - 4
        slot(lf, 0, True, False)
        slot(lf + 1, 1, True, False)
        slot(lf + 2, 2, True, False)
        slot(lf + 3, 3, True, False)
        for kslot in range(2):
            for ci in range(D // 8):
                w_copy(tils[kslot], wsems[kslot], ci, HIST - 1 + kslot).wait()

    img = k(val, table, global_token)
    out = jnp.transpose(img, (2, 4, 0, 1, 3))
    return out.reshape(BATCH, OUT_L, D)


def kernel(val, table, global_token):
    return _sc_embed(val.astype(jnp.int32), table, global_token)


# speedup vs baseline: 1.0318x; 1.0318x over previous
import functools

import jax
import jax.numpy as jnp
from jax import lax
from jax.experimental import pallas as pl
from jax.experimental.pallas import tpu as pltpu
from jax.experimental.pallas import tpu_sc as plsc

BATCH = 4096
HIST = 200
D = 32
OUT_L = HIST + 1
NC = 2
NS = 16
NW = NC * NS
B_PER_W = BATCH // NW
LANES = 16


def _sc_embed(val, table, global_token):
    mesh = plsc.VectorSubcoreMesh(core_axis_name="c", subcore_axis_name="s")

    @functools.partial(
        pl.kernel,
        mesh=mesh,
        out_type=jax.ShapeDtypeStruct((OUT_L, D // 8, NW, 8, B_PER_W), jnp.float32),
        scratch_types=[
            pltpu.VMEM((B_PER_W, HIST), jnp.int32),
            pltpu.VMEM((HIST * B_PER_W,), jnp.int32),
            pltpu.VMEM((B_PER_W, D), jnp.float32),
            pltpu.VMEM((B_PER_W, D), jnp.float32),
            pltpu.VMEM((B_PER_W, D), jnp.float32),
            pltpu.VMEM((B_PER_W, D), jnp.float32),
            pltpu.VMEM((D, B_PER_W), jnp.float32),
            pltpu.VMEM((D, B_PER_W), jnp.float32),
            pltpu.VMEM((D,), jnp.float32),
            pltpu.SemaphoreType.DMA,
            pltpu.SemaphoreType.DMA,
            pltpu.SemaphoreType.DMA,
            pltpu.SemaphoreType.DMA,
            pltpu.SemaphoreType.DMA,
            pltpu.SemaphoreType.DMA,
        ],
        compiler_params=pltpu.CompilerParams(
            use_tc_tiling_on_sc=False, needs_layout_passes=False),
    )
    def k(val_hbm, table_hbm, gt_hbm, out_hbm,
          idx_raw, idx_t, r0, r1, r2, r3, t0, t1, gt_v,
          g0, g1, g2, g3, w0, w1):
        rows = (r0, r1, r2, r3)
        gsems = (g0, g1, g2, g3)
        tils = (t0, t1)
        wsems = (w0, w1)

        wid = lax.axis_index("s") * NC + lax.axis_index("c")
        b0 = wid * B_PER_W

        pltpu.sync_copy(val_hbm.at[pl.ds(b0, B_PER_W)], idx_raw)
        pltpu.sync_copy(gt_hbm.at[0], gt_v)

        iota = lax.iota(jnp.int32, LANES)
        zeros = (iota + wid) // (LANES + NW)

        @plsc.parallel_loop(0, HIST, unroll=2)
        def idx_body(l):
            for j in range(B_PER_W // LANES):
                g = plsc.load_gather(idx_raw, [iota + j * LANES, zeros + l])
                idx_t[pl.ds(l * B_PER_W + j * LANES, LANES)] = g

        for c in range(D):
            bc = plsc.load_gather(gt_v, [zeros + c])
            for j in range(B_PER_W // LANES):
                t0[c, pl.ds(j * LANES, LANES)] = bc
        for ci in range(D // 8):
            pltpu.sync_copy(t0.at[pl.ds(ci * 8, 8)], out_hbm.at[0, ci, wid])

        def g_copy(l, rbuf, gsem):
            return pltpu.make_async_copy(
                table_hbm.at[idx_t.at[pl.ds(l * B_PER_W, B_PER_W)]], rbuf, gsem)

        def w_copy(tbuf, wsem, ci, p):
            return pltpu.make_async_copy(
                tbuf.at[pl.ds(ci * 8, 8)], out_hbm.at[p, ci, wid], wsem)

        def transpose(rbuf, tbuf):
            @plsc.parallel_loop(0, B_PER_W // LANES, unroll=2)
            def j_body(j):
                for c in range(D):
                    g = plsc.load_gather(rbuf, [iota + j * LANES, zeros + c])
                    tbuf[c, pl.ds(j * LANES, LANES)] = g

        def slot(l, kslot, drain_w, prefetch):
            rbuf, gsem = rows[kslot % 4], gsems[kslot % 4]
            tbuf, wsem = tils[kslot % 2], wsems[kslot % 2]
            g_copy(l, rbuf, gsem).wait()
            if drain_w:
                for ci in range(D // 8):
                    w_copy(tbuf, wsem, ci, l - 1).wait()
            transpose(rbuf, tbuf)
            for ci in range(D // 8):
                w_copy(tbuf, wsem, ci, l + 1).start()
            if prefetch:
                g_copy(l + 4, rbuf, gsem).start()

        for l in range(4):
            g_copy(l, rows[l], gsems[l]).start()
        slot(0, 0, False, True)
        slot(1, 1, False, True)
        slot(2, 2, True, True)
        slot(3, 3, True, True)

        def body(q, carry):
            l = 4 * q
            slot(l, 0, True, True)
            slot(l + 1, 1, True, True)
            slot(l + 2, 2, True, True)
            slot(l + 3, 3, True, True)
            return carry

        lax.fori_loop(1, HIST // 4 - 1, body, 0)
        lf = HIST - 4
        slot(lf, 0, True, False)
        slot(lf + 1, 1, True, False)
        slot(lf + 2, 2, True, False)
        slot(lf + 3, 3, True, False)
        for kslot in range(2):
            for ci in range(D // 8):
                w_copy(tils[kslot], wsems[kslot], ci, HIST - 1 + kslot).wait()

    img = k(val, table, global_token)
    out = jnp.transpose(img, (2, 4, 0, 1, 3))
    return out.reshape(BATCH, OUT_L, D)


def kernel(val, table, global_token):
    return _sc_embed(val.astype(jnp.int32), table, global_token)


# speedup vs baseline: 1.0406x; 1.0086x over previous
import functools

import jax
import jax.numpy as jnp
from jax import lax
from jax.experimental import pallas as pl
from jax.experimental.pallas import tpu as pltpu
from jax.experimental.pallas import tpu_sc as plsc

BATCH = 4096
HIST = 200
D = 32
OUT_L = HIST + 1
NC = 2
NS = 16
NW = NC * NS
B_PER_W = BATCH // NW
LANES = 16


def _sc_embed(val, table, global_token):
    mesh = plsc.VectorSubcoreMesh(core_axis_name="c", subcore_axis_name="s")

    @functools.partial(
        pl.kernel,
        mesh=mesh,
        out_type=jax.ShapeDtypeStruct((OUT_L, D // 8, NW, 8, B_PER_W), jnp.float32),
        scratch_types=[
            pltpu.VMEM((B_PER_W, HIST), jnp.int32),
            pltpu.VMEM((HIST * B_PER_W,), jnp.int32),
            pltpu.VMEM((B_PER_W, D), jnp.float32),
            pltpu.VMEM((B_PER_W, D), jnp.float32),
            pltpu.VMEM((B_PER_W, D), jnp.float32),
            pltpu.VMEM((B_PER_W, D), jnp.float32),
            pltpu.VMEM((D // 8, 8, B_PER_W), jnp.float32),
            pltpu.VMEM((D // 8, 8, B_PER_W), jnp.float32),
            pltpu.VMEM((D,), jnp.float32),
            pltpu.SemaphoreType.DMA,
            pltpu.SemaphoreType.DMA,
            pltpu.SemaphoreType.DMA,
            pltpu.SemaphoreType.DMA,
            pltpu.SemaphoreType.DMA,
            pltpu.SemaphoreType.DMA,
        ],
        compiler_params=pltpu.CompilerParams(
            use_tc_tiling_on_sc=False, needs_layout_passes=False),
    )
    def k(val_hbm, table_hbm, gt_hbm, out_hbm,
          idx_raw, idx_t, r0, r1, r2, r3, t0, t1, gt_v,
          g0, g1, g2, g3, w0, w1):
        rows = (r0, r1, r2, r3)
        gsems = (g0, g1, g2, g3)
        tils = (t0, t1)
        wsems = (w0, w1)

        wid = lax.axis_index("s") * NC + lax.axis_index("c")
        b0 = wid * B_PER_W

        pltpu.sync_copy(val_hbm.at[pl.ds(b0, B_PER_W)], idx_raw)
        pltpu.sync_copy(gt_hbm.at[0], gt_v)

        iota = lax.iota(jnp.int32, LANES)
        zeros = (iota + wid) // (LANES + NW)

        @plsc.parallel_loop(0, HIST, unroll=2)
        def idx_body(l):
            for j in range(B_PER_W // LANES):
                g = plsc.load_gather(idx_raw, [iota + j * LANES, zeros + l])
                idx_t[pl.ds(l * B_PER_W + j * LANES, LANES)] = g

        for c in range(D):
            bc = plsc.load_gather(gt_v, [zeros + c])
            for j in range(B_PER_W // LANES):
                t0[c // 8, c % 8, pl.ds(j * LANES, LANES)] = bc
        pltpu.sync_copy(t0, out_hbm.at[0, pl.ds(0, D // 8), wid])

        def g_copy(l, rbuf, gsem):
            return pltpu.make_async_copy(
                table_hbm.at[idx_t.at[pl.ds(l * B_PER_W, B_PER_W)]], rbuf, gsem)

        def w_copy(tbuf, wsem, p):
            return pltpu.make_async_copy(
                tbuf, out_hbm.at[p, pl.ds(0, D // 8), wid], wsem)

        def transpose(rbuf, tbuf):
            @plsc.parallel_loop(0, B_PER_W // LANES, unroll=2)
            def j_body(j):
                for c in range(D):
                    g = plsc.load_gather(rbuf, [iota + j * LANES, zeros + c])
                    tbuf[c // 8, c % 8, pl.ds(j * LANES, LANES)] = g

        def slot(l, kslot, drain_w, prefetch):
            rbuf, gsem = rows[kslot % 4], gsems[kslot % 4]
            tbuf, wsem = tils[kslot % 2], wsems[kslot % 2]
            g_copy(l, rbuf, gsem).wait()
            if drain_w:
                w_copy(tbuf, wsem, l - 1).wait()
            transpose(rbuf, tbuf)
            w_copy(tbuf, wsem, l + 1).start()
            if prefetch:
                g_copy(l + 4, rbuf, gsem).start()

        for l in range(4):
            g_copy(l, rows[l], gsems[l]).start()
        slot(0, 0, False, True)
        slot(1, 1, False, True)
        slot(2, 2, True, True)
        slot(3, 3, True, True)

        def body(q, carry):
            l = 4 * q
            slot(l, 0, True, True)
            slot(l + 1, 1, True, True)
            slot(l + 2, 2, True, True)
            slot(l + 3, 3, True, True)
            return carry

        lax.fori_loop(1, HIST // 4 - 1, body, 0)
        lf = HIST - 4
        slot(lf, 0, True, False)
        slot(lf + 1, 1, True, False)
        slot(lf + 2, 2, True, False)
        slot(lf + 3, 3, True, False)
        for kslot in range(2):
            w_copy(tils[kslot], wsems[kslot], HIST - 1 + kslot).wait()

    img = k(val, table, global_token)
    out = jnp.transpose(img, (2, 4, 0, 1, 3))
    return out.reshape(BATCH, OUT_L, D)


def kernel(val, table, global_token):
    return _sc_embed(val.astype(jnp.int32), table, global_token)
